# baseline (speedup 1.0000x reference)
import functools

import jax
import jax.numpy as jnp
import numpy as np
from jax import lax
from jax.experimental import pallas as pl
from jax.experimental.pallas import tpu as pltpu
from jax.experimental.pallas import tpu_sc as plsc

N_NODES = 10000
N_EDGES = 320000
D_FEAT = 128
NC = 2
NS = 16
DH = D_FEAT // NC
NGRP = DH // 16
N_PAD = 10112
ROWS_PER_SUB = N_PAD // NS
E_PER_SUB = N_EDGES // NS
BLK = 79
NBLK = ROWS_PER_SUB // BLK
CHUNK = 80
CH_PER_SUB = E_PER_SUB // CHUNK
NBUF = 5
NCH = 10
NGROUP = NCH // NBUF
N_IBLK = CH_PER_SUB // NCH

_PERM32 = np.empty(32, np.int64)
_PERM32[0::2] = np.arange(16)
_PERM32[1::2] = np.arange(16) + 16
_PERM64 = np.concatenate([_PERM32, _PERM32 + 32])


def _sc_body(tabp_bf, tab, row2d, col2d, tr2d, out, t1, Tb, A, b0, b1, bb,
             rbf0, rbf1, rbf2, rbf3, rbf4,
             sb0, sb1, sb2, sb3, sb4,
             rblk, cblk, tvb,
             g0s, g1s, g2s, g3s, g4s, s0s, s1s, s2s, s3s, s4s):
    rbf = [rbf0, rbf1, rbf2, rbf3, rbf4]
    sbuf = [sb0, sb1, sb2, sb3, sb4]
    gsem = [g0s, g1s, g2s, g3s, g4s]
    ssem = [s0s, s1s, s2s, s3s, s4s]
    c = lax.axis_index("c")
    s = lax.axis_index("s")
    r0 = s * ROWS_PER_SUB
    g0 = c * N_PAD + r0

    def zrow(r, _):
        for j in range(NGRP):
            b1[r, pl.ds(16 * j, 16)] = jnp.zeros((16,), jnp.float32)
        return _
    lax.fori_loop(0, BLK, zrow, None)
    for k in range(NBLK):
        sl = pl.ds(r0 + k * BLK, BLK)
        gl = pl.ds(g0 + k * BLK, BLK)
        pltpu.sync_copy(tabp_bf.at[gl], Tb.at[sl])
        pltpu.sync_copy(b1, A.at[sl])
    plsc.subcore_barrier()

    def layer():
        def scale_chunk(jj, gbuf, obuf):
            def scale(e, _):
                t16 = plsc.load_gather(
                    tvb, [jnp.full((16,), jj, jnp.int32),
                          jnp.full((16,), e, jnp.int32)])
                for h in range(2):
                    v = gbuf[e, pl.ds(32 * h, 32)]
                    va, vb = plsc.unpack(v, format=plsc.PackFormat.INTERLEAVED)
                    obuf[e, pl.ds(32 * h, 16)] = va * t16
                    obuf[e, pl.ds(32 * h + 16, 16)] = vb * t16
                return _
            lax.fori_loop(0, CHUNK, scale, None, unroll=4)

        def iblk_body(b, _):
            ch0 = s * CH_PER_SUB + b * NCH
            pltpu.sync_copy(row2d.at[pl.ds(ch0, NCH)], rblk)
            pltpu.sync_copy(col2d.at[pl.ds(ch0, NCH)], cblk)
            pltpu.sync_copy(tr2d.at[pl.ds(ch0, NCH)], tvb)
            for k in range(NBUF):
                pltpu.async_copy(Tb.at[rblk.at[k]], rbf[k], gsem[k])
            for g in range(NGROUP):
                for k in range(NBUF):
                    j = g * NBUF + k
                    pltpu.make_async_copy(Tb.at[rblk.at[j]], rbf[k],
                                          gsem[k]).wait()
                    if g > 0:
                        pltpu.make_async_copy(sbuf[k],
                                              A.at[cblk.at[j - NBUF]],
                                              ssem[k]).wait()
                    scale_chunk(j, rbf[k], sbuf[k])
                    pltpu.async_copy(sbuf[k], A.at[cblk.at[j]], ssem[k],
                                     add=True)
                    if g + 1 < NGROUP:
                        pltpu.async_copy(Tb.at[rblk.at[j + NBUF]], rbf[k],
                                         gsem[k])
            for k in range(NBUF):
                j = (NGROUP - 1) * NBUF + k
                pltpu.make_async_copy(sbuf[k], A.at[cblk.at[j]],
                                      ssem[k]).wait()
            return _
        lax.fori_loop(0, N_IBLK, iblk_body, None)

    layer()
    plsc.subcore_barrier()
    def packrow(r, _):
        for h in range(2):
            a16 = b0[r, pl.ds(32 * h, 16)]
            b16 = b0[r, pl.ds(32 * h + 16, 16)]
            bb[r, pl.ds(32 * h, 32)] = plsc.pack(
                a16, b16, format=plsc.PackFormat.INTERLEAVED)
        return _
    for k in range(NBLK):
        sl = pl.ds(r0 + k * BLK, BLK)
        gl = pl.ds(g0 + k * BLK, BLK)
        pltpu.sync_copy(A.at[sl], b0)
        pltpu.sync_copy(b0, t1.at[gl])
        lax.fori_loop(0, BLK, packrow, None, unroll=4)
        pltpu.sync_copy(bb, Tb.at[sl])
        pltpu.sync_copy(b1, A.at[sl])
    plsc.subcore_barrier()
    layer()
    plsc.subcore_barrier()

    third = jnp.full((16,), 1.0 / 3.0, jnp.float32)
    def add1(r, _):
        for j in range(NGRP):
            d = pl.ds(16 * j, 16)
            b0[r, d] = b0[r, d] + b1[r, d]
        return _
    def add2(r, _):
        for j in range(NGRP):
            d = pl.ds(16 * j, 16)
            b0[r, d] = (b0[r, d] + b1[r, d]) * third
        return _
    for k in range(NBLK):
        sl = pl.ds(r0 + k * BLK, BLK)
        gl = pl.ds(g0 + k * BLK, BLK)
        pltpu.sync_copy(tab.at[gl], b0)
        pltpu.sync_copy(t1.at[gl], b1)
        lax.fori_loop(0, BLK, add1, None, unroll=4)
        pltpu.sync_copy(A.at[sl], b1)
        lax.fori_loop(0, BLK, add2, None, unroll=4)
        pltpu.sync_copy(b0, out.at[gl])


_sc_kernel = functools.partial(
    pl.kernel,
    out_type=jax.ShapeDtypeStruct((NC * N_PAD, DH), jnp.float32),
    mesh=plsc.VectorSubcoreMesh(core_axis_name="c", subcore_axis_name="s"),
    compiler_params=pltpu.CompilerParams(
        needs_layout_passes=False, use_tc_tiling_on_sc=False),
    scratch_types=[
        pltpu.HBM((NC * N_PAD, DH), jnp.float32),
        pltpu.VMEM_SHARED((N_PAD, DH), jnp.bfloat16),
        pltpu.VMEM_SHARED((N_PAD, DH), jnp.float32),
        pltpu.VMEM((BLK, DH), jnp.float32),
        pltpu.VMEM((BLK, DH), jnp.float32),
        pltpu.VMEM((BLK, DH), jnp.bfloat16),
        pltpu.VMEM((CHUNK, DH), jnp.bfloat16),
        pltpu.VMEM((CHUNK, DH), jnp.bfloat16),
        pltpu.VMEM((CHUNK, DH), jnp.bfloat16),
        pltpu.VMEM((CHUNK, DH), jnp.bfloat16),
        pltpu.VMEM((CHUNK, DH), jnp.bfloat16),
        pltpu.VMEM((CHUNK, DH), jnp.float32),
        pltpu.VMEM((CHUNK, DH), jnp.float32),
        pltpu.VMEM((CHUNK, DH), jnp.float32),
        pltpu.VMEM((CHUNK, DH), jnp.float32),
        pltpu.VMEM((CHUNK, DH), jnp.float32),
        pltpu.VMEM((NCH, CHUNK), jnp.int32),
        pltpu.VMEM((NCH, CHUNK), jnp.int32),
        pltpu.VMEM((NCH, CHUNK), jnp.float32),
        pltpu.SemaphoreType.DMA,
        pltpu.SemaphoreType.DMA,
        pltpu.SemaphoreType.DMA,
        pltpu.SemaphoreType.DMA,
        pltpu.SemaphoreType.DMA,
        pltpu.SemaphoreType.DMA,
        pltpu.SemaphoreType.DMA,
        pltpu.SemaphoreType.DMA,
        pltpu.SemaphoreType.DMA,
        pltpu.SemaphoreType.DMA,
    ],
)(_sc_body)


def kernel(embed, edge_index, trend):
    row = edge_index[0].astype(jnp.int32)
    col = edge_index[1].astype(jnp.int32)
    e_pad = jnp.pad(embed, ((0, N_PAD - N_NODES), (0, 0)))
    tab = e_pad.reshape(N_PAD, NC, DH).transpose(1, 0, 2).reshape(NC * N_PAD, DH)
    tabp_bf = tab[:, _PERM64].astype(jnp.bfloat16)
    row2d = row.reshape(N_EDGES // CHUNK, CHUNK)
    col2d = col.reshape(N_EDGES // CHUNK, CHUNK)
    tr2d = trend.astype(jnp.float32).reshape(N_EDGES // CHUNK, CHUNK)
    out = _sc_kernel(tabp_bf, tab, row2d, col2d, tr2d)
    out = out.reshape(NC, N_PAD, DH).transpose(1, 0, 2).reshape(N_PAD, D_FEAT)
    return out[:N_NODES]


# speedup vs baseline: 1.0916x; 1.0916x over previous
import functools

import jax
import jax.numpy as jnp
import numpy as np
from jax import lax
from jax.experimental import pallas as pl
from jax.experimental.pallas import tpu as pltpu
from jax.experimental.pallas import tpu_sc as plsc

N_NODES = 10000
N_EDGES = 320000
D_FEAT = 128
NC = 2
NS = 16
DH = D_FEAT // NC
NGRP = DH // 16
N_PAD = 10112
ROWS_PER_SUB = N_PAD // NS
E_PER_SUB = N_EDGES // NS
BLK = 79
NBLK = ROWS_PER_SUB // BLK
CHUNK = 80
CH_PER_SUB = E_PER_SUB // CHUNK
NBUF = 5
NCH = 25
NGROUP = NCH // NBUF
N_IBLK = CH_PER_SUB // NCH

_PERM32 = np.empty(32, np.int64)
_PERM32[0::2] = np.arange(16)
_PERM32[1::2] = np.arange(16) + 16
_PERM64 = np.concatenate([_PERM32, _PERM32 + 32])


def _sc_body(tabp_bf, tab, row2d, col2d, tr2d, out, t1, Tb, A, b0, b1, bb,
             rbf0, rbf1, rbf2, rbf3, rbf4,
             sb0, sb1, sb2, sb3, sb4,
             rblk, cblk, tvb,
             g0s, g1s, g2s, g3s, g4s, s0s, s1s, s2s, s3s, s4s):
    rbf = [rbf0, rbf1, rbf2, rbf3, rbf4]
    sbuf = [sb0, sb1, sb2, sb3, sb4]
    gsem = [g0s, g1s, g2s, g3s, g4s]
    ssem = [s0s, s1s, s2s, s3s, s4s]
    c = lax.axis_index("c")
    s = lax.axis_index("s")
    r0 = s * ROWS_PER_SUB
    g0 = c * N_PAD + r0

    def zrow(r, _):
        for j in range(NGRP):
            b1[r, pl.ds(16 * j, 16)] = jnp.zeros((16,), jnp.float32)
        return _
    lax.fori_loop(0, BLK, zrow, None)
    for k in range(NBLK):
        sl = pl.ds(r0 + k * BLK, BLK)
        gl = pl.ds(g0 + k * BLK, BLK)
        pltpu.sync_copy(tabp_bf.at[gl], Tb.at[sl])
        pltpu.sync_copy(b1, A.at[sl])
    plsc.subcore_barrier()

    def layer():
        def scale_chunk(jj, gbuf, obuf):
            def scale(e, _):
                t16 = plsc.load_gather(
                    tvb, [jnp.full((16,), jj, jnp.int32),
                          jnp.full((16,), e, jnp.int32)])
                for h in range(2):
                    v = gbuf[e, pl.ds(32 * h, 32)]
                    w = plsc.bitcast(v, jnp.int32)
                    va = plsc.bitcast(lax.shift_left(w, 16), jnp.float32)
                    vb = plsc.bitcast(
                        w & jnp.full((16,), -65536, jnp.int32), jnp.float32)
                    obuf[e, pl.ds(32 * h, 16)] = va * t16
                    obuf[e, pl.ds(32 * h + 16, 16)] = vb * t16
                return _
            lax.fori_loop(0, CHUNK, scale, None, unroll=4)

        def iblk_body(b, _):
            ch0 = s * CH_PER_SUB + b * NCH
            pltpu.sync_copy(row2d.at[pl.ds(ch0, NCH)], rblk)
            pltpu.sync_copy(col2d.at[pl.ds(ch0, NCH)], cblk)
            pltpu.sync_copy(tr2d.at[pl.ds(ch0, NCH)], tvb)
            for k in range(NBUF):
                pltpu.async_copy(Tb.at[rblk.at[k]], rbf[k], gsem[k])
            def group_body(g, _):
                for k in range(NBUF):
                    j = g * NBUF + k
                    pltpu.make_async_copy(Tb.at[rblk.at[j]], rbf[k],
                                          gsem[k]).wait()
                    @pl.when(g > 0)
                    def _wait_prev():
                        pltpu.make_async_copy(sbuf[k],
                                              A.at[cblk.at[j - NBUF]],
                                              ssem[k]).wait()
                    scale_chunk(j, rbf[k], sbuf[k])
                    pltpu.async_copy(sbuf[k], A.at[cblk.at[j]], ssem[k],
                                     add=True)
                    @pl.when(g + 1 < NGROUP)
                    def _start_next():
                        pltpu.async_copy(Tb.at[rblk.at[j + NBUF]], rbf[k],
                                         gsem[k])
                return _
            lax.fori_loop(0, NGROUP, group_body, None)
            for k in range(NBUF):
                j = (NGROUP - 1) * NBUF + k
                pltpu.make_async_copy(sbuf[k], A.at[cblk.at[j]],
                                      ssem[k]).wait()
            return _
        lax.fori_loop(0, N_IBLK, iblk_body, None)

    layer()
    plsc.subcore_barrier()
    def packrow(r, _):
        for h in range(2):
            a16 = b0[r, pl.ds(32 * h, 16)]
            b16 = b0[r, pl.ds(32 * h + 16, 16)]
            bb[r, pl.ds(32 * h, 32)] = plsc.pack(
                a16, b16, format=plsc.PackFormat.INTERLEAVED)
        return _
    for k in range(NBLK):
        sl = pl.ds(r0 + k * BLK, BLK)
        gl = pl.ds(g0 + k * BLK, BLK)
        pltpu.sync_copy(A.at[sl], b0)
        pltpu.sync_copy(b0, t1.at[gl])
        lax.fori_loop(0, BLK, packrow, None, unroll=4)
        pltpu.sync_copy(bb, Tb.at[sl])
        pltpu.sync_copy(b1, A.at[sl])
    plsc.subcore_barrier()
    layer()
    plsc.subcore_barrier()

    third = jnp.full((16,), 1.0 / 3.0, jnp.float32)
    def add1(r, _):
        for j in range(NGRP):
            d = pl.ds(16 * j, 16)
            b0[r, d] = b0[r, d] + b1[r, d]
        return _
    def add2(r, _):
        for j in range(NGRP):
            d = pl.ds(16 * j, 16)
            b0[r, d] = (b0[r, d] + b1[r, d]) * third
        return _
    for k in range(NBLK):
        sl = pl.ds(r0 + k * BLK, BLK)
        gl = pl.ds(g0 + k * BLK, BLK)
        pltpu.sync_copy(tab.at[gl], b0)
        pltpu.sync_copy(t1.at[gl], b1)
        lax.fori_loop(0, BLK, add1, None, unroll=4)
        pltpu.sync_copy(A.at[sl], b1)
        lax.fori_loop(0, BLK, add2, None, unroll=4)
        pltpu.sync_copy(b0, out.at[gl])


_sc_kernel = functools.partial(
    pl.kernel,
    out_type=jax.ShapeDtypeStruct((NC * N_PAD, DH), jnp.float32),
    mesh=plsc.VectorSubcoreMesh(core_axis_name="c", subcore_axis_name="s"),
    compiler_params=pltpu.CompilerParams(
        needs_layout_passes=False, use_tc_tiling_on_sc=False),
    scratch_types=[
        pltpu.HBM((NC * N_PAD, DH), jnp.float32),
        pltpu.VMEM_SHARED((N_PAD, DH), jnp.bfloat16),
        pltpu.VMEM_SHARED((N_PAD, DH), jnp.float32),
        pltpu.VMEM((BLK, DH), jnp.float32),
        pltpu.VMEM((BLK, DH), jnp.float32),
        pltpu.VMEM((BLK, DH), jnp.bfloat16),
        pltpu.VMEM((CHUNK, DH), jnp.bfloat16),
        pltpu.VMEM((CHUNK, DH), jnp.bfloat16),
        pltpu.VMEM((CHUNK, DH), jnp.bfloat16),
        pltpu.VMEM((CHUNK, DH), jnp.bfloat16),
        pltpu.VMEM((CHUNK, DH), jnp.bfloat16),
        pltpu.VMEM((CHUNK, DH), jnp.float32),
        pltpu.VMEM((CHUNK, DH), jnp.float32),
        pltpu.VMEM((CHUNK, DH), jnp.float32),
        pltpu.VMEM((CHUNK, DH), jnp.float32),
        pltpu.VMEM((CHUNK, DH), jnp.float32),
        pltpu.VMEM((NCH, CHUNK), jnp.int32),
        pltpu.VMEM((NCH, CHUNK), jnp.int32),
        pltpu.VMEM((NCH, CHUNK), jnp.float32),
        pltpu.SemaphoreType.DMA,
        pltpu.SemaphoreType.DMA,
        pltpu.SemaphoreType.DMA,
        pltpu.SemaphoreType.DMA,
        pltpu.SemaphoreType.DMA,
        pltpu.SemaphoreType.DMA,
        pltpu.SemaphoreType.DMA,
        pltpu.SemaphoreType.DMA,
        pltpu.SemaphoreType.DMA,
        pltpu.SemaphoreType.DMA,
    ],
)(_sc_body)


def kernel(embed, edge_index, trend):
    row = edge_index[0].astype(jnp.int32)
    col = edge_index[1].astype(jnp.int32)
    e_pad = jnp.pad(embed, ((0, N_PAD - N_NODES), (0, 0)))
    tab = e_pad.reshape(N_PAD, NC, DH).transpose(1, 0, 2).reshape(NC * N_PAD, DH)
    tabp_bf = tab[:, _PERM64].astype(jnp.bfloat16)
    row2d = row.reshape(N_EDGES // CHUNK, CHUNK)
    col2d = col.reshape(N_EDGES // CHUNK, CHUNK)
    tr2d = trend.astype(jnp.float32).reshape(N_EDGES // CHUNK, CHUNK)
    out = _sc_kernel(tabp_bf, tab, row2d, col2d, tr2d)
    out = out.reshape(NC, N_PAD, DH).transpose(1, 0, 2).reshape(N_PAD, D_FEAT)
    return out[:N_NODES]


# speedup vs baseline: 1.4203x; 1.3011x over previous
import functools

import jax
import jax.numpy as jnp
from jax import lax
from jax.experimental import pallas as pl
from jax.experimental.pallas import tpu as pltpu
from jax.experimental.pallas import tpu_sc as plsc

N_NODES = 10000
N_EDGES = 320000
D_FEAT = 128
NC = 2
NS = 16
DH = D_FEAT // NC
NGRP = DH // 16
N_PAD = 10112
ROWS_PER_SUB = N_PAD // NS
E_PER_SUB = N_EDGES // NS
BLK = 79
NBLK = ROWS_PER_SUB // BLK
CHUNK = 80
CH_PER_SUB = E_PER_SUB // CHUNK
NBUF = 5
NCH = 25
NGROUP = NCH // NBUF
N_IBLK = CH_PER_SUB // NCH


def _sc_body(tab, idx3, out, T, A, b0, b1,
             rows0, rows1, rows2, rows3, rows4,
             i3,
             g0s, g1s, g2s, g3s, g4s, s0s, s1s, s2s, s3s, s4s):
    rows = [rows0, rows1, rows2, rows3, rows4]
    gsem = [g0s, g1s, g2s, g3s, g4s]
    ssem = [s0s, s1s, s2s, s3s, s4s]
    c = lax.axis_index("c")
    s = lax.axis_index("s")
    r0 = s * ROWS_PER_SUB
    g0 = c * N_PAD + r0

    def zrow(r, _):
        for j in range(NGRP):
            b1[r, pl.ds(16 * j, 16)] = jnp.zeros((16,), jnp.float32)
        return _
    lax.fori_loop(0, BLK, zrow, None)
    for k in range(NBLK):
        pltpu.sync_copy(tab.at[pl.ds(g0 + k * BLK, BLK)], b0)
        pltpu.sync_copy(b0, T.at[pl.ds(r0 + k * BLK, BLK)])
        pltpu.sync_copy(b1, A.at[pl.ds(r0 + k * BLK, BLK)])
    plsc.subcore_barrier()

    def layer(src, acc):
        def scale_chunk(jj, buf):
            def scale(e, _):
                t16 = plsc.bitcast(plsc.load_gather(
                    i3, [jnp.full((16,), jj, jnp.int32),
                         jnp.full((16,), 2, jnp.int32),
                         jnp.full((16,), e, jnp.int32)]), jnp.float32)
                for j in range(NGRP):
                    d = pl.ds(16 * j, 16)
                    buf[e, d] = buf[e, d] * t16
                return _
            lax.fori_loop(0, CHUNK, scale, None, unroll=4)

        def iblk_body(b, _):
            ch0 = s * CH_PER_SUB + b * NCH
            pltpu.sync_copy(idx3.at[pl.ds(ch0, NCH)], i3)
            for k in range(NBUF):
                pltpu.async_copy(src.at[i3.at[k, 0]], rows[k], gsem[k])
            for g in range(NGROUP):
                for k in range(NBUF):
                    j = g * NBUF + k
                    pltpu.make_async_copy(src.at[i3.at[j, 0]], rows[k],
                                          gsem[k]).wait()
                    scale_chunk(j, rows[k])
                    pltpu.async_copy(rows[k], acc.at[i3.at[j, 1]], ssem[k],
                                     add=True)
                for k in range(NBUF):
                    j = g * NBUF + k
                    pltpu.make_async_copy(rows[k], acc.at[i3.at[j, 1]],
                                          ssem[k]).wait()
                    if g + 1 < NGROUP:
                        pltpu.async_copy(src.at[i3.at[j + NBUF, 0]], rows[k],
                                         gsem[k])
            return _
        lax.fori_loop(0, N_IBLK, iblk_body, None)

    layer(T, A)
    plsc.subcore_barrier()
    for k in range(NBLK):
        pltpu.sync_copy(b1, T.at[pl.ds(r0 + k * BLK, BLK)])
    plsc.subcore_barrier()
    layer(A, T)
    plsc.subcore_barrier()

    third = jnp.full((16,), 1.0 / 3.0, jnp.float32)
    def add1(r, _):
        for j in range(NGRP):
            d = pl.ds(16 * j, 16)
            b0[r, d] = b0[r, d] + b1[r, d]
        return _
    def add2(r, _):
        for j in range(NGRP):
            d = pl.ds(16 * j, 16)
            b0[r, d] = (b0[r, d] + b1[r, d]) * third
        return _
    for k in range(NBLK):
        pltpu.sync_copy(tab.at[pl.ds(g0 + k * BLK, BLK)], b0)
        pltpu.sync_copy(A.at[pl.ds(r0 + k * BLK, BLK)], b1)
        lax.fori_loop(0, BLK, add1, None, unroll=4)
        pltpu.sync_copy(T.at[pl.ds(r0 + k * BLK, BLK)], b1)
        lax.fori_loop(0, BLK, add2, None, unroll=4)
        pltpu.sync_copy(b0, out.at[pl.ds(g0 + k * BLK, BLK)])


_sc_kernel = functools.partial(
    pl.kernel,
    out_type=jax.ShapeDtypeStruct((NC * N_PAD, DH), jnp.float32),
    mesh=plsc.VectorSubcoreMesh(core_axis_name="c", subcore_axis_name="s"),
    compiler_params=pltpu.CompilerParams(
        needs_layout_passes=False, use_tc_tiling_on_sc=False),
    scratch_types=[
        pltpu.VMEM_SHARED((N_PAD, DH), jnp.float32),
        pltpu.VMEM_SHARED((N_PAD, DH), jnp.float32),
        pltpu.VMEM((BLK, DH), jnp.float32),
        pltpu.VMEM((BLK, DH), jnp.float32),
        pltpu.VMEM((CHUNK, DH), jnp.float32),
        pltpu.VMEM((CHUNK, DH), jnp.float32),
        pltpu.VMEM((CHUNK, DH), jnp.float32),
        pltpu.VMEM((CHUNK, DH), jnp.float32),
        pltpu.VMEM((CHUNK, DH), jnp.float32),
        pltpu.VMEM((NCH, 3, CHUNK), jnp.int32),
        pltpu.SemaphoreType.DMA,
        pltpu.SemaphoreType.DMA,
        pltpu.SemaphoreType.DMA,
        pltpu.SemaphoreType.DMA,
        pltpu.SemaphoreType.DMA,
        pltpu.SemaphoreType.DMA,
        pltpu.SemaphoreType.DMA,
        pltpu.SemaphoreType.DMA,
        pltpu.SemaphoreType.DMA,
        pltpu.SemaphoreType.DMA,
    ],
)(_sc_body)


def kernel(embed, edge_index, trend):
    row = edge_index[0].astype(jnp.int32)
    col = edge_index[1].astype(jnp.int32)
    e_pad = jnp.pad(embed, ((0, N_PAD - N_NODES), (0, 0)))
    tab = e_pad.reshape(N_PAD, NC, DH).transpose(1, 0, 2).reshape(NC * N_PAD, DH)
    row2d = row.reshape(N_EDGES // CHUNK, CHUNK)
    col2d = col.reshape(N_EDGES // CHUNK, CHUNK)
    tr2d = lax.bitcast_convert_type(
        trend.astype(jnp.float32), jnp.int32).reshape(N_EDGES // CHUNK, CHUNK)
    idx3 = jnp.stack([row2d, col2d, tr2d], axis=1)
    out = _sc_kernel(tab, idx3)
    out = out.reshape(NC, N_PAD, DH).transpose(1, 0, 2).reshape(N_PAD, D_FEAT)
    return out[:N_NODES]


# speedup vs baseline: 1.4432x; 1.0161x over previous
import functools

import jax
import jax.numpy as jnp
from jax import lax
from jax.experimental import pallas as pl
from jax.experimental.pallas import tpu as pltpu
from jax.experimental.pallas import tpu_sc as plsc

N_NODES = 10000
N_EDGES = 320000
D_FEAT = 128
NC = 2
NS = 16
DH = D_FEAT // NC
NGRP = DH // 16
N_PAD = 10112
ROWS_PER_SUB = N_PAD // NS
E_PER_SUB = N_EDGES // NS
BLK = 79
NBLK = ROWS_PER_SUB // BLK
CHUNK = 80
CH_PER_SUB = E_PER_SUB // CHUNK
NBUF = 5
NCH = 25
NGROUP = NCH // NBUF
N_IBLK = CH_PER_SUB // NCH


def _sc_body(tab, row2d, col2d, tr2d, out, T, A, b0, b1,
             rows0, rows1, rows2, rows3, rows4,
             rblk, cblk, tvb,
             g0s, g1s, g2s, g3s, g4s, s0s, s1s, s2s, s3s, s4s):
    rows = [rows0, rows1, rows2, rows3, rows4]
    gsem = [g0s, g1s, g2s, g3s, g4s]
    ssem = [s0s, s1s, s2s, s3s, s4s]
    c = lax.axis_index("c")
    s = lax.axis_index("s")
    r0 = s * ROWS_PER_SUB
    g0 = c * N_PAD + r0

    def zrow(r, _):
        for j in range(NGRP):
            b1[r, pl.ds(16 * j, 16)] = jnp.zeros((16,), jnp.float32)
        return _
    lax.fori_loop(0, BLK, zrow, None)
    for k in range(NBLK):
        pltpu.sync_copy(tab.at[pl.ds(g0 + k * BLK, BLK)], b0)
        pltpu.sync_copy(b0, T.at[pl.ds(r0 + k * BLK, BLK)])
        pltpu.sync_copy(b1, A.at[pl.ds(r0 + k * BLK, BLK)])
    plsc.subcore_barrier()

    def layer(src, acc):
        def scale_chunk(jj, buf):
            def scale(e, _):
                t16 = plsc.load_gather(
                    tvb, [jnp.full((16,), jj, jnp.int32),
                          jnp.full((16,), e, jnp.int32)])
                for j in range(NGRP):
                    d = pl.ds(16 * j, 16)
                    buf[e, d] = buf[e, d] * t16
                return _
            lax.fori_loop(0, CHUNK, scale, None, unroll=4)

        def iblk_body(b, _):
            ch0 = s * CH_PER_SUB + b * NCH
            pltpu.sync_copy(row2d.at[pl.ds(ch0, NCH)], rblk)
            pltpu.sync_copy(col2d.at[pl.ds(ch0, NCH)], cblk)
            pltpu.sync_copy(tr2d.at[pl.ds(ch0, NCH)], tvb)
            for k in range(NBUF):
                pltpu.async_copy(src.at[rblk.at[k]], rows[k], gsem[k])
            for g in range(NGROUP):
                for k in range(NBUF):
                    j = g * NBUF + k
                    pltpu.make_async_copy(src.at[rblk.at[j]], rows[k],
                                          gsem[k]).wait()
                    scale_chunk(j, rows[k])
                    pltpu.async_copy(rows[k], acc.at[cblk.at[j]], ssem[k],
                                     add=True)
                for k in range(NBUF):
                    j = g * NBUF + k
                    pltpu.make_async_copy(rows[k], acc.at[cblk.at[j]],
                                          ssem[k]).wait()
                    if g + 1 < NGROUP:
                        pltpu.async_copy(src.at[rblk.at[j + NBUF]], rows[k],
                                         gsem[k])
            return _
        lax.fori_loop(0, N_IBLK, iblk_body, None)

    layer(T, A)
    plsc.subcore_barrier()
    for k in range(NBLK):
        pltpu.sync_copy(b1, T.at[pl.ds(r0 + k * BLK, BLK)])
    plsc.subcore_barrier()
    layer(A, T)
    plsc.subcore_barrier()

    third = jnp.full((16,), 1.0 / 3.0, jnp.float32)
    def add1(r, _):
        for j in range(NGRP):
            d = pl.ds(16 * j, 16)
            b0[r, d] = b0[r, d] + b1[r, d]
        return _
    def add2(r, _):
        for j in range(NGRP):
            d = pl.ds(16 * j, 16)
            b0[r, d] = (b0[r, d] + b1[r, d]) * third
        return _
    for k in range(NBLK):
        pltpu.sync_copy(tab.at[pl.ds(g0 + k * BLK, BLK)], b0)
        pltpu.sync_copy(A.at[pl.ds(r0 + k * BLK, BLK)], b1)
        lax.fori_loop(0, BLK, add1, None, unroll=4)
        pltpu.sync_copy(T.at[pl.ds(r0 + k * BLK, BLK)], b1)
        lax.fori_loop(0, BLK, add2, None, unroll=4)
        pltpu.sync_copy(b0, out.at[pl.ds(g0 + k * BLK, BLK)])


_sc_kernel = functools.partial(
    pl.kernel,
    out_type=jax.ShapeDtypeStruct((NC * N_PAD, DH), jnp.float32),
    mesh=plsc.VectorSubcoreMesh(core_axis_name="c", subcore_axis_name="s"),
    compiler_params=pltpu.CompilerParams(
        needs_layout_passes=False, use_tc_tiling_on_sc=False),
    scratch_types=[
        pltpu.VMEM_SHARED((N_PAD, DH), jnp.float32),
        pltpu.VMEM_SHARED((N_PAD, DH), jnp.float32),
        pltpu.VMEM((BLK, DH), jnp.float32),
        pltpu.VMEM((BLK, DH), jnp.float32),
        pltpu.VMEM((CHUNK, DH), jnp.float32),
        pltpu.VMEM((CHUNK, DH), jnp.float32),
        pltpu.VMEM((CHUNK, DH), jnp.float32),
        pltpu.VMEM((CHUNK, DH), jnp.float32),
        pltpu.VMEM((CHUNK, DH), jnp.float32),
        pltpu.VMEM((NCH, CHUNK), jnp.int32),
        pltpu.VMEM((NCH, CHUNK), jnp.int32),
        pltpu.VMEM((NCH, CHUNK), jnp.float32),
        pltpu.SemaphoreType.DMA,
        pltpu.SemaphoreType.DMA,
        pltpu.SemaphoreType.DMA,
        pltpu.SemaphoreType.DMA,
        pltpu.SemaphoreType.DMA,
        pltpu.SemaphoreType.DMA,
        pltpu.SemaphoreType.DMA,
        pltpu.SemaphoreType.DMA,
        pltpu.SemaphoreType.DMA,
        pltpu.SemaphoreType.DMA,
    ],
)(_sc_body)


def kernel(embed, edge_index, trend):
    row = edge_index[0].astype(jnp.int32)
    col = edge_index[1].astype(jnp.int32)
    e_pad = jnp.pad(embed, ((0, N_PAD - N_NODES), (0, 0)))
    tab = e_pad.reshape(N_PAD, NC, DH).transpose(1, 0, 2).reshape(NC * N_PAD, DH)
    row2d = row.reshape(N_EDGES // CHUNK, CHUNK)
    col2d = col.reshape(N_EDGES // CHUNK, CHUNK)
    tr2d = trend.astype(jnp.float32).reshape(N_EDGES // CHUNK, CHUNK)
    out = _sc_kernel(tab, row2d, col2d, tr2d)
    out = out.reshape(NC, N_PAD, DH).transpose(1, 0, 2).reshape(N_PAD, D_FEAT)
    return out[:N_NODES]
